# Initial kernel scaffold; baseline (speedup 1.0000x reference)
import jax
import jax.numpy as jnp
from jax.experimental import pallas as pl


def kernel(x, edge_index, batch, W_e0, a_src_e0, a_dst_e0, b_e0, W_e1, a_src_e1, a_dst_e1, b_e1, Wg1, bg1, Wg2, bg2, W_d0, a_src_d0, a_dst_d0, b_d0, W_d1, a_src_d1, a_dst_d1, b_d1):
    raise NotImplementedError("write your pallas kernel here")


# speedup vs baseline: 2.6113x; 2.6113x over previous
import functools

import jax
import jax.numpy as jnp
from jax.experimental import pallas as pl

_N = 10000
_G = 16
_ROW_BLK = 2000


def _mm_body(x_ref, w_ref, b_ref, o_ref, *, relu):
    o = jnp.dot(x_ref[:], w_ref[:], preferred_element_type=jnp.float32)
    o = o + b_ref[:]
    if relu:
        o = jnp.maximum(o, 0.0)
    o_ref[:] = o


def _mm(x, w, b=None, relu=False):
    n, k = x.shape
    m = w.shape[1]
    if b is None:
        b = jnp.zeros((1, m), jnp.float32)
    else:
        b = b.reshape(1, m)
    blk = _ROW_BLK if n % _ROW_BLK == 0 else n
    grid = n // blk
    return pl.pallas_call(
        functools.partial(_mm_body, relu=relu),
        grid=(grid,),
        in_specs=[
            pl.BlockSpec((blk, k), lambda i: (i, 0)),
            pl.BlockSpec((k, m), lambda i: (0, 0)),
            pl.BlockSpec((1, m), lambda i: (0, 0)),
        ],
        out_specs=pl.BlockSpec((blk, m), lambda i: (i, 0)),
        out_shape=jax.ShapeDtypeStruct((n, m), jnp.float32),
    )(x, w, b)


def _edge_e_body(es_ref, ed_ref, o_ref):
    e = es_ref[:] + ed_ref[:]
    o_ref[:] = jnp.where(e >= 0.0, e, 0.2 * e)


def _edge_e(es, ed):
    n, h = es.shape
    blk = _ROW_BLK
    return pl.pallas_call(
        _edge_e_body,
        grid=(n // blk,),
        in_specs=[
            pl.BlockSpec((blk, h), lambda i: (i, 0)),
            pl.BlockSpec((blk, h), lambda i: (i, 0)),
        ],
        out_specs=pl.BlockSpec((blk, h), lambda i: (i, 0)),
        out_shape=jax.ShapeDtypeStruct((n, h), jnp.float32),
    )(es, ed)


def _edge_p_body(e_ref, md_ref, o_ref):
    o_ref[:] = jnp.exp(e_ref[:] - md_ref[:])


def _edge_p(e, md):
    n, h = e.shape
    blk = _ROW_BLK
    return pl.pallas_call(
        _edge_p_body,
        grid=(n // blk,),
        in_specs=[
            pl.BlockSpec((blk, h), lambda i: (i, 0)),
            pl.BlockSpec((blk, h), lambda i: (i, 0)),
        ],
        out_specs=pl.BlockSpec((blk, h), lambda i: (i, 0)),
        out_shape=jax.ShapeDtypeStruct((n, h), jnp.float32),
    )(e, md)


def _edge_msg_body(hs_ref, p_ref, zd_ref, r_ref, o_ref):
    a = p_ref[:] / (zd_ref[:] + 1e-16)
    o_ref[:] = hs_ref[:] * jnp.dot(a, r_ref[:], preferred_element_type=jnp.float32)


def _edge_msg(h_src, p, zd, r):
    n, f = h_src.shape
    h = p.shape[1]
    blk = _ROW_BLK
    return pl.pallas_call(
        _edge_msg_body,
        grid=(n // blk,),
        in_specs=[
            pl.BlockSpec((blk, f), lambda i: (i, 0)),
            pl.BlockSpec((blk, h), lambda i: (i, 0)),
            pl.BlockSpec((blk, h), lambda i: (i, 0)),
            pl.BlockSpec((h, f), lambda i: (0, 0)),
        ],
        out_specs=pl.BlockSpec((blk, f), lambda i: (i, 0)),
        out_shape=jax.ShapeDtypeStruct((n, f), jnp.float32),
    )(h_src, p, zd, r)


def _gat_layer(x, src, dst, w, a_s, a_d, b, heads, oc, relu_out):
    n = x.shape[0]
    h = _mm(x, w)

    a_s = a_s.reshape(heads, oc)
    a_d = a_d.reshape(heads, oc)
    eye_h = jnp.eye(heads, dtype=jnp.float32)
    proj_s = (a_s[:, :, None] * eye_h[:, None, :]).reshape(heads * oc, heads)
    proj_d = (a_d[:, :, None] * eye_h[:, None, :]).reshape(heads * oc, heads)
    s_src = _mm(h, proj_s)
    s_dst = _mm(h, proj_d)

    e = _edge_e(s_src[src], s_dst[dst])
    m = jax.ops.segment_max(e, dst, num_segments=n)
    m = jnp.where(jnp.isfinite(m), m, 0.0)
    p = _edge_p(e, m[dst])
    z = jax.ops.segment_sum(p, dst, num_segments=n)

    r = jnp.repeat(eye_h, oc, axis=1)
    msg = _edge_msg(h[src], p, z[dst], r)
    o = jax.ops.segment_sum(msg, dst, num_segments=n)

    mean_m = (jnp.ones((heads, 1, 1), jnp.float32)
              * jnp.eye(oc, dtype=jnp.float32)[None] / heads).reshape(heads * oc, oc)
    return _mm(o, mean_m, b=b, relu=relu_out)


def _pool(x, batch, wg1, bg1, wg2, bg2):
    g = _mm(x, wg1, b=bg1, relu=True)
    g = _mm(g, wg2, b=bg2)
    m = jax.ops.segment_max(g, batch, num_segments=_G)
    p = _edge_p(g, m[batch])
    z = jax.ops.segment_sum(p, batch, num_segments=_G)
    ones_r = jnp.ones((1, x.shape[1]), jnp.float32)
    weighted = _edge_msg(x, p, z[batch], ones_r)
    return jax.ops.segment_sum(weighted, batch, num_segments=_G)


def kernel(x, edge_index, batch, W_e0, a_src_e0, a_dst_e0, b_e0, W_e1, a_src_e1,
           a_dst_e1, b_e1, Wg1, bg1, Wg2, bg2, W_d0, a_src_d0, a_dst_d0, b_d0,
           W_d1, a_src_d1, a_dst_d1, b_d1):
    n = x.shape[0]
    loops = jnp.arange(n)
    src = jnp.concatenate([edge_index[0], loops])
    dst = jnp.concatenate([edge_index[1], loops])

    h = _gat_layer(x, src, dst, W_e0, a_src_e0, a_dst_e0, b_e0, 8, 128, True)
    h = _gat_layer(h, src, dst, W_e1, a_src_e1, a_dst_e1, b_e1, 8, 64, False)
    pooled = _pool(h, batch, Wg1, bg1, Wg2, bg2)
    h = pooled[batch]
    h = _gat_layer(h, src, dst, W_d0, a_src_d0, a_dst_d0, b_d0, 1, 128, True)
    h = _gat_layer(h, src, dst, W_d1, a_src_d1, a_dst_d1, b_d1, 1, 128, False)
    return h


# speedup vs baseline: 2.9336x; 1.1234x over previous
import functools

import jax
import jax.numpy as jnp
from jax.experimental import pallas as pl

_N = 10000
_G = 16
_ROW_BLK = 2000


def _mm_body(x_ref, w_ref, b_ref, o_ref, *, relu):
    o = jnp.dot(x_ref[:], w_ref[:], preferred_element_type=jnp.float32)
    o = o + b_ref[:]
    if relu:
        o = jnp.maximum(o, 0.0)
    o_ref[:] = o


def _mm(x, w, b=None, relu=False):
    n, k = x.shape
    m = w.shape[1]
    if b is None:
        b = jnp.zeros((1, m), jnp.float32)
    else:
        b = b.reshape(1, m)
    blk = _ROW_BLK if n % _ROW_BLK == 0 else n
    grid = n // blk
    return pl.pallas_call(
        functools.partial(_mm_body, relu=relu),
        grid=(grid,),
        in_specs=[
            pl.BlockSpec((blk, k), lambda i: (i, 0)),
            pl.BlockSpec((k, m), lambda i: (0, 0)),
            pl.BlockSpec((1, m), lambda i: (0, 0)),
        ],
        out_specs=pl.BlockSpec((blk, m), lambda i: (i, 0)),
        out_shape=jax.ShapeDtypeStruct((n, m), jnp.float32),
    )(x, w, b)


def _edge_e_body(es_ref, ed_ref, o_ref):
    e = es_ref[:] + ed_ref[:]
    o_ref[:] = jnp.where(e >= 0.0, e, 0.2 * e)


def _edge_e(es, ed):
    n, h = es.shape
    blk = _ROW_BLK
    return pl.pallas_call(
        _edge_e_body,
        grid=(n // blk,),
        in_specs=[
            pl.BlockSpec((blk, h), lambda i: (i, 0)),
            pl.BlockSpec((blk, h), lambda i: (i, 0)),
        ],
        out_specs=pl.BlockSpec((blk, h), lambda i: (i, 0)),
        out_shape=jax.ShapeDtypeStruct((n, h), jnp.float32),
    )(es, ed)


def _edge_p_body(e_ref, md_ref, o_ref):
    o_ref[:] = jnp.exp(e_ref[:] - md_ref[:])


def _edge_p(e, md):
    n, h = e.shape
    blk = _ROW_BLK
    return pl.pallas_call(
        _edge_p_body,
        grid=(n // blk,),
        in_specs=[
            pl.BlockSpec((blk, h), lambda i: (i, 0)),
            pl.BlockSpec((blk, h), lambda i: (i, 0)),
        ],
        out_specs=pl.BlockSpec((blk, h), lambda i: (i, 0)),
        out_shape=jax.ShapeDtypeStruct((n, h), jnp.float32),
    )(e, md)


def _edge_msg_body(hs_ref, p_ref, zd_ref, o_ref, *, heads, oc):
    a = p_ref[:] / (zd_ref[:] + 1e-16)
    hs = hs_ref[:]
    acc = hs[:, 0:oc] * a[:, 0:1]
    for hd in range(1, heads):
        acc = acc + hs[:, hd * oc:(hd + 1) * oc] * a[:, hd:hd + 1]
    o_ref[:] = acc * (1.0 / heads)


def _edge_msg(h_src, p, zd, heads, oc):
    n = h_src.shape[0]
    blk = _ROW_BLK
    return pl.pallas_call(
        functools.partial(_edge_msg_body, heads=heads, oc=oc),
        grid=(n // blk,),
        in_specs=[
            pl.BlockSpec((blk, heads * oc), lambda i: (i, 0)),
            pl.BlockSpec((blk, heads), lambda i: (i, 0)),
            pl.BlockSpec((blk, heads), lambda i: (i, 0)),
        ],
        out_specs=pl.BlockSpec((blk, oc), lambda i: (i, 0)),
        out_shape=jax.ShapeDtypeStruct((n, oc), jnp.float32),
    )(h_src, p, zd)


def _gat_layer(x, src, dst, w, a_s, a_d, b, heads, oc, relu_out):
    n = x.shape[0]
    h = _mm(x, w)

    a_s = a_s.reshape(heads, oc)
    a_d = a_d.reshape(heads, oc)
    eye_h = jnp.eye(heads, dtype=jnp.float32)
    proj_s = (a_s[:, :, None] * eye_h[:, None, :]).reshape(heads * oc, heads)
    proj_d = (a_d[:, :, None] * eye_h[:, None, :]).reshape(heads * oc, heads)
    s_src = _mm(h, proj_s)
    s_dst = _mm(h, proj_d)

    e = _edge_e(s_src[src], s_dst[dst])
    m = jax.ops.segment_max(e, dst, num_segments=n)
    m = jnp.where(jnp.isfinite(m), m, 0.0)
    p = _edge_p(e, m[dst])
    z = jax.ops.segment_sum(p, dst, num_segments=n)

    msg = _edge_msg(h[src], p, z[dst], heads, oc)
    o = jax.ops.segment_sum(msg, dst, num_segments=n)
    return _mm(o, jnp.eye(oc, dtype=jnp.float32), b=b, relu=relu_out)


def _pool(x, batch, wg1, bg1, wg2, bg2):
    g = _mm(x, wg1, b=bg1, relu=True)
    g = _mm(g, wg2, b=bg2)
    m = jax.ops.segment_max(g, batch, num_segments=_G)
    p = _edge_p(g, m[batch])
    z = jax.ops.segment_sum(p, batch, num_segments=_G)
    weighted = _edge_msg(x, p, z[batch], 1, x.shape[1])
    return jax.ops.segment_sum(weighted, batch, num_segments=_G)


def kernel(x, edge_index, batch, W_e0, a_src_e0, a_dst_e0, b_e0, W_e1, a_src_e1,
           a_dst_e1, b_e1, Wg1, bg1, Wg2, bg2, W_d0, a_src_d0, a_dst_d0, b_d0,
           W_d1, a_src_d1, a_dst_d1, b_d1):
    n = x.shape[0]
    loops = jnp.arange(n)
    src = jnp.concatenate([edge_index[0], loops])
    dst = jnp.concatenate([edge_index[1], loops])

    h = _gat_layer(x, src, dst, W_e0, a_src_e0, a_dst_e0, b_e0, 8, 128, True)
    h = _gat_layer(h, src, dst, W_e1, a_src_e1, a_dst_e1, b_e1, 8, 64, False)
    pooled = _pool(h, batch, Wg1, bg1, Wg2, bg2)
    h = pooled[batch]
    h = _gat_layer(h, src, dst, W_d0, a_src_d0, a_dst_d0, b_d0, 1, 128, True)
    h = _gat_layer(h, src, dst, W_d1, a_src_d1, a_dst_d1, b_d1, 1, 128, False)
    return h
